# baseline (speedup 1.0000x reference)
import jax
import jax.numpy as jnp
from jax import lax
from jax.experimental import pallas as pl
from jax.experimental.pallas import tpu as pltpu
from jax.experimental.pallas import tpu_sc as plsc

NC = 2
NS = 16
NW = NC * NS
LANES = 16
BATCH = 16384
D = 64
DP = 128
CHUNK = 128
B_PER_W = BATCH // NW
N_CHUNKS = B_PER_W // CHUNK
IDX_COLS = 128
IDX_ROWS = BATCH // IDX_COLS
ROWS_PER_W = N_CHUNKS
COL_CHUNKS = D // LANES
GROUPS = CHUNK // LANES


def _sc_body(h_hbm, r_hbm, t_hbm, wen_hbm, wre_hbm, out_hbm,
             hidx, ridx, tidx, hpair, rpair, tpair, hoff, roff, toff,
             hbuf, rbuf, tbuf, acc, sem_i, sem_h, sem_r, sem_t):
    wid = lax.axis_index("s") * NC + lax.axis_index("c")
    acc[...] = jnp.zeros((LANES,), jnp.float32)
    base = wid * ROWS_PER_W
    ci_h = pltpu.async_copy(h_hbm.at[pl.ds(base, ROWS_PER_W)], hidx, sem_i)
    ci_r = pltpu.async_copy(r_hbm.at[pl.ds(base, ROWS_PER_W)], ridx, sem_h)
    ci_t = pltpu.async_copy(t_hbm.at[pl.ds(base, ROWS_PER_W)], tidx, sem_r)
    ci_h.wait()
    ci_r.wait()
    ci_t.wait()

    @pl.loop(0, ROWS_PER_W)
    def _(j):
        @pl.loop(0, IDX_COLS // LANES)
        def _(q):
            sl = pl.ds(q * LANES, LANES)
            for idx, pair, off in ((hidx, hpair, hoff),
                                   (ridx, rpair, roff),
                                   (tidx, tpair, toff)):
                v = idx[j, sl]
                pair[j, sl] = v >> 1
                off[j, sl] = (v & 1) * D

    for j in range(N_CHUNKS):
        ch = pltpu.async_copy(wen_hbm.at[hpair.at[j]], hbuf, sem_h)
        cr = pltpu.async_copy(wre_hbm.at[rpair.at[j]], rbuf, sem_r)
        ct = pltpu.async_copy(wen_hbm.at[tpair.at[j]], tbuf, sem_t)
        ch.wait()
        cr.wait()
        ct.wait()

        @pl.loop(0, GROUPS)
        def _(g):
            gs = pl.ds(g * LANES, LANES)
            hv = hoff[j, gs]
            rv = roff[j, gs]
            tv = toff[j, gs]
            for k in range(LANES):
                row = g * LANES + k
                ho = hv[k]
                ro = rv[k]
                to = tv[k]
                for c in range(COL_CHUNKS):
                    v = (hbuf[row, pl.ds(ho + c * LANES, LANES)]
                         + rbuf[row, pl.ds(ro + c * LANES, LANES)]
                         - tbuf[row, pl.ds(to + c * LANES, LANES)])
                    acc[...] += v * v

    pltpu.sync_copy(acc, out_hbm.at[wid])


_sc_gather_reduce = pl.kernel(
    _sc_body,
    out_type=jax.ShapeDtypeStruct((NW, LANES), jnp.float32),
    mesh=plsc.VectorSubcoreMesh(core_axis_name="c", subcore_axis_name="s"),
    scratch_types=[
        pltpu.VMEM((ROWS_PER_W, IDX_COLS), jnp.int32),
        pltpu.VMEM((ROWS_PER_W, IDX_COLS), jnp.int32),
        pltpu.VMEM((ROWS_PER_W, IDX_COLS), jnp.int32),
        pltpu.VMEM((ROWS_PER_W, IDX_COLS), jnp.int32),
        pltpu.VMEM((ROWS_PER_W, IDX_COLS), jnp.int32),
        pltpu.VMEM((ROWS_PER_W, IDX_COLS), jnp.int32),
        pltpu.VMEM((ROWS_PER_W, IDX_COLS), jnp.int32),
        pltpu.VMEM((ROWS_PER_W, IDX_COLS), jnp.int32),
        pltpu.VMEM((ROWS_PER_W, IDX_COLS), jnp.int32),
        pltpu.VMEM((CHUNK, DP), jnp.float32),
        pltpu.VMEM((CHUNK, DP), jnp.float32),
        pltpu.VMEM((CHUNK, DP), jnp.float32),
        pltpu.VMEM((LANES,), jnp.float32),
        pltpu.SemaphoreType.DMA,
        pltpu.SemaphoreType.DMA,
        pltpu.SemaphoreType.DMA,
        pltpu.SemaphoreType.DMA,
    ],
)


def _tc_reduce_body(p_ref, o_ref):
    o_ref[0, 0] = jnp.sum(p_ref[...])


def kernel(pos_h, pos_r, pos_t, neg_h, neg_r, neg_t, W_en, W_re):
    del neg_h, neg_r, neg_t
    h2 = pos_h.reshape(IDX_ROWS, IDX_COLS)
    r2 = pos_r.reshape(IDX_ROWS, IDX_COLS)
    t2 = pos_t.reshape(IDX_ROWS, IDX_COLS)
    wen2 = W_en.reshape(W_en.shape[0] // 2, DP)
    wre2 = W_re.reshape(W_re.shape[0] // 2, DP)
    partials = _sc_gather_reduce(h2, r2, t2, wen2, wre2)
    total = pl.pallas_call(
        _tc_reduce_body,
        out_shape=jax.ShapeDtypeStruct((1, 1), jnp.float32),
        out_specs=pl.BlockSpec(memory_space=pltpu.SMEM),
    )(partials)
    return total[0, 0]


# speedup vs baseline: 2.2590x; 2.2590x over previous
import jax
import jax.numpy as jnp
from jax import lax
from jax.experimental import pallas as pl
from jax.experimental.pallas import tpu as pltpu
from jax.experimental.pallas import tpu_sc as plsc

NC = 2
NS = 16
NW = NC * NS
LANES = 16
BATCH = 16384
D = 64
DP = 128
CHUNK = 128
B_PER_W = BATCH // NW
N_CHUNKS = B_PER_W // CHUNK
IDX_COLS = 128
IDX_ROWS = BATCH // IDX_COLS
ROWS_PER_W = N_CHUNKS
COL_CHUNKS = D // LANES
GROUPS = CHUNK // LANES


def _sc_body(h_hbm, r_hbm, t_hbm, wen_hbm, wre_hbm, out_hbm,
             hidx, ridx, tidx, hpair, rpair, tpair, hoff, roff, toff,
             hbuf, rbuf, tbuf, acc, sem_i, sem_h, sem_r, sem_t):
    wid = lax.axis_index("s") * NC + lax.axis_index("c")
    acc[...] = jnp.zeros((LANES,), jnp.float32)
    base = wid * ROWS_PER_W
    ci_h = pltpu.async_copy(h_hbm.at[pl.ds(base, ROWS_PER_W)], hidx, sem_i)
    ci_r = pltpu.async_copy(r_hbm.at[pl.ds(base, ROWS_PER_W)], ridx, sem_h)
    ci_t = pltpu.async_copy(t_hbm.at[pl.ds(base, ROWS_PER_W)], tidx, sem_r)
    ci_h.wait()
    ci_r.wait()
    ci_t.wait()

    @pl.loop(0, ROWS_PER_W)
    def _(j):
        @pl.loop(0, IDX_COLS // LANES)
        def _(q):
            sl = pl.ds(q * LANES, LANES)
            for idx, pair, off in ((hidx, hpair, hoff),
                                   (tidx, tpair, toff)):
                v = idx[j, sl]
                pair[j, sl] = v & (HALF - 1)
                off[j, sl] = (v >> LOG2H) * D
            v = ridx[j, sl]
            rpair[j, sl] = v >> 1
            roff[j, sl] = (v & 1) * D

    for j in range(N_CHUNKS):
        ch = pltpu.async_copy(wen_hbm.at[hpair.at[j]], hbuf, sem_h)
        cr = pltpu.async_copy(wre_hbm.at[rpair.at[j]], rbuf, sem_r)
        ct = pltpu.async_copy(wen_hbm.at[tpair.at[j]], tbuf, sem_t)
        ch.wait()
        cr.wait()
        ct.wait()

        @pl.loop(0, GROUPS)
        def _(g):
            gs = pl.ds(g * LANES, LANES)
            hv = hoff[j, gs]
            rv = roff[j, gs]
            tv = toff[j, gs]
            for k in range(LANES):
                row = g * LANES + k
                ho = hv[k]
                ro = rv[k]
                to = tv[k]
                for c in range(COL_CHUNKS):
                    v = (hbuf[row, pl.ds(ho + c * LANES, LANES)]
                         + rbuf[row, pl.ds(ro + c * LANES, LANES)]
                         - tbuf[row, pl.ds(to + c * LANES, LANES)])
                    acc[...] += v * v

    pltpu.sync_copy(acc, out_hbm.at[wid])


_sc_gather_reduce = pl.kernel(
    _sc_body,
    out_type=jax.ShapeDtypeStruct((NW, LANES), jnp.float32),
    mesh=plsc.VectorSubcoreMesh(core_axis_name="c", subcore_axis_name="s"),
    scratch_types=[
        pltpu.VMEM((ROWS_PER_W, IDX_COLS), jnp.int32),
        pltpu.VMEM((ROWS_PER_W, IDX_COLS), jnp.int32),
        pltpu.VMEM((ROWS_PER_W, IDX_COLS), jnp.int32),
        pltpu.VMEM((ROWS_PER_W, IDX_COLS), jnp.int32),
        pltpu.VMEM((ROWS_PER_W, IDX_COLS), jnp.int32),
        pltpu.VMEM((ROWS_PER_W, IDX_COLS), jnp.int32),
        pltpu.VMEM((ROWS_PER_W, IDX_COLS), jnp.int32),
        pltpu.VMEM((ROWS_PER_W, IDX_COLS), jnp.int32),
        pltpu.VMEM((ROWS_PER_W, IDX_COLS), jnp.int32),
        pltpu.VMEM((CHUNK, DP), jnp.float32),
        pltpu.VMEM((CHUNK, DP), jnp.float32),
        pltpu.VMEM((CHUNK, DP), jnp.float32),
        pltpu.VMEM((LANES,), jnp.float32),
        pltpu.SemaphoreType.DMA,
        pltpu.SemaphoreType.DMA,
        pltpu.SemaphoreType.DMA,
        pltpu.SemaphoreType.DMA,
    ],
)


def _tc_reduce_body(p_ref, o_ref):
    o_ref[0, 0] = jnp.sum(p_ref[...])


TBLK = 8192


HALF = 524288
LOG2H = 19


def _tc_pairize_body(a_ref, b_ref, o_ref):
    o_ref[:, :D] = jnp.transpose(a_ref[...], (1, 0))
    o_ref[:, D:] = jnp.transpose(b_ref[...], (1, 0))


def _tc_pairize(wt):
    n = wt.shape[1]
    nb = HALF // TBLK
    last_full = n // TBLK - 1
    return pl.pallas_call(
        _tc_pairize_body,
        grid=(nb,),
        in_specs=[pl.BlockSpec((D, TBLK), lambda j: (0, j)),
                  pl.BlockSpec((D, TBLK),
                               lambda j: (0, jnp.minimum(j + nb, last_full)))],
        out_specs=pl.BlockSpec((TBLK, DP), lambda j: (j, 0)),
        out_shape=jax.ShapeDtypeStruct((HALF, DP), jnp.float32),
    )(wt, wt)


def kernel(pos_h, pos_r, pos_t, neg_h, neg_r, neg_t, W_en, W_re):
    del neg_h, neg_r, neg_t
    h2 = pos_h.reshape(IDX_ROWS, IDX_COLS)
    r2 = pos_r.reshape(IDX_ROWS, IDX_COLS)
    t2 = pos_t.reshape(IDX_ROWS, IDX_COLS)
    wen2 = _tc_pairize(W_en.T)
    n = W_en.shape[0]
    row0 = (n // TBLK - 1 - HALF // TBLK + 1) * TBLK
    tail = (n - HALF) - row0
    patch = jnp.concatenate(
        [lax.dynamic_slice_in_dim(W_en, row0, tail, 0),
         lax.dynamic_slice_in_dim(W_en, row0 + HALF, tail, 0)], axis=1)
    wen2 = lax.dynamic_update_slice(wen2, patch, (row0, 0))
    wre2 = W_re.reshape(W_re.shape[0] // 2, DP)
    partials = _sc_gather_reduce(h2, r2, t2, wen2, wre2)
    total = pl.pallas_call(
        _tc_reduce_body,
        out_shape=jax.ShapeDtypeStruct((1, 1), jnp.float32),
        out_specs=pl.BlockSpec(memory_space=pltpu.SMEM),
    )(partials)
    return total[0, 0]


# speedup vs baseline: 2.3720x; 1.0500x over previous
import jax
import jax.numpy as jnp
from jax import lax
from jax.experimental import pallas as pl
from jax.experimental.pallas import tpu as pltpu
from jax.experimental.pallas import tpu_sc as plsc

NC = 2
NS = 16
NW = NC * NS
LANES = 16
BATCH = 16384
D = 64
DP = 128
CHUNK = 128
B_PER_W = BATCH // NW
N_CHUNKS = B_PER_W // CHUNK
IDX_COLS = 128
IDX_ROWS = BATCH // IDX_COLS
ROWS_PER_W = N_CHUNKS
COL_CHUNKS = D // LANES
GROUPS = CHUNK // LANES


def _sc_body(h_hbm, r_hbm, t_hbm, wen_hbm, wre_hbm, out_hbm,
             hidx, ridx, tidx, hpair, rpair, tpair, hoff, roff, toff,
             hbuf, rbuf, tbuf, acc, sem_i, sem_h, sem_r, sem_t):
    wid = lax.axis_index("s") * NC + lax.axis_index("c")
    acc[...] = jnp.zeros((LANES,), jnp.float32)
    base = wid * ROWS_PER_W
    ci_h = pltpu.async_copy(h_hbm.at[pl.ds(base, ROWS_PER_W)], hidx, sem_i)
    ci_r = pltpu.async_copy(r_hbm.at[pl.ds(base, ROWS_PER_W)], ridx, sem_h)
    ci_t = pltpu.async_copy(t_hbm.at[pl.ds(base, ROWS_PER_W)], tidx, sem_r)
    ci_h.wait()
    ci_r.wait()
    ci_t.wait()

    @pl.loop(0, ROWS_PER_W)
    def _(j):
        @pl.loop(0, IDX_COLS // LANES)
        def _(q):
            sl = pl.ds(q * LANES, LANES)
            for idx, pair, off in ((hidx, hpair, hoff),
                                   (tidx, tpair, toff)):
                v = idx[j, sl]
                pair[j, sl] = v & (HALF - 1)
                off[j, sl] = (v >> LOG2H) * D
            v = ridx[j, sl]
            rpair[j, sl] = v >> 1
            roff[j, sl] = (v & 1) * D

    for j in range(N_CHUNKS):
        ch = pltpu.async_copy(wen_hbm.at[hpair.at[j]], hbuf, sem_h)
        cr = pltpu.async_copy(wre_hbm.at[rpair.at[j]], rbuf, sem_r)
        ct = pltpu.async_copy(wen_hbm.at[tpair.at[j]], tbuf, sem_t)
        ch.wait()
        cr.wait()
        ct.wait()

        @pl.loop(0, GROUPS)
        def _(g):
            gs = pl.ds(g * LANES, LANES)
            hv = hoff[j, gs]
            rv = roff[j, gs]
            tv = toff[j, gs]
            for k in range(LANES):
                row = g * LANES + k
                ho = hv[k]
                ro = rv[k]
                to = tv[k]
                for c in range(COL_CHUNKS):
                    v = (hbuf[row, pl.ds(ho + c * LANES, LANES)]
                         + rbuf[row, pl.ds(ro + c * LANES, LANES)]
                         - tbuf[row, pl.ds(to + c * LANES, LANES)])
                    acc[...] += v * v

    pltpu.sync_copy(acc, out_hbm.at[wid])


_sc_gather_reduce = pl.kernel(
    _sc_body,
    out_type=jax.ShapeDtypeStruct((NW, LANES), jnp.float32),
    mesh=plsc.VectorSubcoreMesh(core_axis_name="c", subcore_axis_name="s"),
    scratch_types=[
        pltpu.VMEM((ROWS_PER_W, IDX_COLS), jnp.int32),
        pltpu.VMEM((ROWS_PER_W, IDX_COLS), jnp.int32),
        pltpu.VMEM((ROWS_PER_W, IDX_COLS), jnp.int32),
        pltpu.VMEM((ROWS_PER_W, IDX_COLS), jnp.int32),
        pltpu.VMEM((ROWS_PER_W, IDX_COLS), jnp.int32),
        pltpu.VMEM((ROWS_PER_W, IDX_COLS), jnp.int32),
        pltpu.VMEM((ROWS_PER_W, IDX_COLS), jnp.int32),
        pltpu.VMEM((ROWS_PER_W, IDX_COLS), jnp.int32),
        pltpu.VMEM((ROWS_PER_W, IDX_COLS), jnp.int32),
        pltpu.VMEM((CHUNK, DP), jnp.float32),
        pltpu.VMEM((CHUNK, DP), jnp.float32),
        pltpu.VMEM((CHUNK, DP), jnp.float32),
        pltpu.VMEM((LANES,), jnp.float32),
        pltpu.SemaphoreType.DMA,
        pltpu.SemaphoreType.DMA,
        pltpu.SemaphoreType.DMA,
        pltpu.SemaphoreType.DMA,
    ],
)


def _tc_reduce_body(p_ref, o_ref):
    o_ref[0, 0] = jnp.sum(p_ref[...])


TBLK = 16384


HALF = 524288
LOG2H = 19


def _tc_pairize_body(a_ref, b_ref, o_ref):
    o_ref[:, :D] = jnp.transpose(a_ref[...], (1, 0))
    o_ref[:, D:] = jnp.transpose(b_ref[...], (1, 0))


def _tc_pairize(wt):
    n = wt.shape[1]
    nb = HALF // TBLK
    last_full = n // TBLK - 1
    return pl.pallas_call(
        _tc_pairize_body,
        grid=(nb,),
        in_specs=[pl.BlockSpec((D, TBLK), lambda j: (0, j)),
                  pl.BlockSpec((D, TBLK),
                               lambda j: (0, jnp.minimum(j + nb, last_full)))],
        out_specs=pl.BlockSpec((TBLK, DP), lambda j: (j, 0)),
        out_shape=jax.ShapeDtypeStruct((HALF, DP), jnp.float32),
    )(wt, wt)


def kernel(pos_h, pos_r, pos_t, neg_h, neg_r, neg_t, W_en, W_re):
    del neg_h, neg_r, neg_t
    h2 = pos_h.reshape(IDX_ROWS, IDX_COLS)
    r2 = pos_r.reshape(IDX_ROWS, IDX_COLS)
    t2 = pos_t.reshape(IDX_ROWS, IDX_COLS)
    wen2 = _tc_pairize(W_en.T)
    n = W_en.shape[0]
    row0 = (n // TBLK - 1 - HALF // TBLK + 1) * TBLK
    tail = (n - HALF) - row0
    patch = jnp.concatenate(
        [lax.dynamic_slice_in_dim(W_en, row0, tail, 0),
         lax.dynamic_slice_in_dim(W_en, row0 + HALF, tail, 0)], axis=1)
    wen2 = lax.dynamic_update_slice(wen2, patch, (row0, 0))
    wre2 = W_re.reshape(W_re.shape[0] // 2, DP)
    partials = _sc_gather_reduce(h2, r2, t2, wen2, wre2)
    total = pl.pallas_call(
        _tc_reduce_body,
        out_shape=jax.ShapeDtypeStruct((1, 1), jnp.float32),
        out_specs=pl.BlockSpec(memory_space=pltpu.SMEM),
    )(partials)
    return total[0, 0]


# speedup vs baseline: 2.4301x; 1.0245x over previous
import jax
import jax.numpy as jnp
from jax import lax
from jax.experimental import pallas as pl
from jax.experimental.pallas import tpu as pltpu
from jax.experimental.pallas import tpu_sc as plsc

NC = 2
NS = 16
NW = NC * NS
LANES = 16
BATCH = 16384
D = 64
DP = 128
CHUNK = 128
B_PER_W = BATCH // NW
N_CHUNKS = B_PER_W // CHUNK
IDX_COLS = 128
IDX_ROWS = BATCH // IDX_COLS
ROWS_PER_W = N_CHUNKS
COL_CHUNKS = D // LANES
GROUPS = CHUNK // LANES


def _sc_body(h_hbm, r_hbm, t_hbm, wen_hbm, wre_hbm, out_hbm,
             hidx, ridx, tidx, hpair, rpair, tpair, hoff, roff, toff,
             hbuf0, rbuf0, tbuf0, hbuf1, rbuf1, tbuf1, acc,
             sem_i, sem0, sem1):
    wid = lax.axis_index("s") * NC + lax.axis_index("c")
    acc[...] = jnp.zeros((LANES,), jnp.float32)
    base = wid * ROWS_PER_W
    ci_h = pltpu.async_copy(h_hbm.at[pl.ds(base, ROWS_PER_W)], hidx, sem_i)
    ci_r = pltpu.async_copy(r_hbm.at[pl.ds(base, ROWS_PER_W)], ridx, sem0)
    ci_t = pltpu.async_copy(t_hbm.at[pl.ds(base, ROWS_PER_W)], tidx, sem1)
    ci_h.wait()
    ci_r.wait()
    ci_t.wait()

    @pl.loop(0, ROWS_PER_W)
    def _(j):
        @pl.loop(0, IDX_COLS // LANES)
        def _(q):
            sl = pl.ds(q * LANES, LANES)
            for idx, pair, off in ((hidx, hpair, hoff),
                                   (tidx, tpair, toff)):
                v = idx[j, sl]
                pair[j, sl] = v & (HALF - 1)
                off[j, sl] = (v >> LOG2H) * D
            v = ridx[j, sl]
            rpair[j, sl] = v >> 1
            roff[j, sl] = (v & 1) * D

    bufs = ((hbuf0, rbuf0, tbuf0, sem0), (hbuf1, rbuf1, tbuf1, sem1))

    def fire(j, bset):
        hb, rb, tb, sem = bset
        return (pltpu.async_copy(wen_hbm.at[hpair.at[j]], hb, sem),
                pltpu.async_copy(wre_hbm.at[rpair.at[j]], rb, sem),
                pltpu.async_copy(wen_hbm.at[tpair.at[j]], tb, sem))

    def compute(j, bset):
        hb, rb, tb, _ = bset

        @pl.loop(0, GROUPS)
        def _(g):
            gs = pl.ds(g * LANES, LANES)
            hv = hoff[j, gs]
            rv = roff[j, gs]
            tv = toff[j, gs]
            for k in range(LANES):
                row = g * LANES + k
                ho = hv[k]
                ro = rv[k]
                to = tv[k]
                for c in range(COL_CHUNKS):
                    v = (hb[row, pl.ds(ho + c * LANES, LANES)]
                         + rb[row, pl.ds(ro + c * LANES, LANES)]
                         - tb[row, pl.ds(to + c * LANES, LANES)])
                    acc[...] += v * v

    pending = fire(0, bufs[0])
    for j in range(N_CHUNKS):
        nxt = fire(j + 1, bufs[(j + 1) % 2]) if j + 1 < N_CHUNKS else None
        for cp in pending:
            cp.wait()
        compute(j, bufs[j % 2])
        pending = nxt

    pltpu.sync_copy(acc, out_hbm.at[wid])


_sc_gather_reduce = pl.kernel(
    _sc_body,
    out_type=jax.ShapeDtypeStruct((NW, LANES), jnp.float32),
    mesh=plsc.VectorSubcoreMesh(core_axis_name="c", subcore_axis_name="s"),
    scratch_types=[
        pltpu.VMEM((ROWS_PER_W, IDX_COLS), jnp.int32),
        pltpu.VMEM((ROWS_PER_W, IDX_COLS), jnp.int32),
        pltpu.VMEM((ROWS_PER_W, IDX_COLS), jnp.int32),
        pltpu.VMEM((ROWS_PER_W, IDX_COLS), jnp.int32),
        pltpu.VMEM((ROWS_PER_W, IDX_COLS), jnp.int32),
        pltpu.VMEM((ROWS_PER_W, IDX_COLS), jnp.int32),
        pltpu.VMEM((ROWS_PER_W, IDX_COLS), jnp.int32),
        pltpu.VMEM((ROWS_PER_W, IDX_COLS), jnp.int32),
        pltpu.VMEM((ROWS_PER_W, IDX_COLS), jnp.int32),
        pltpu.VMEM((CHUNK, DP), jnp.float32),
        pltpu.VMEM((CHUNK, DP), jnp.float32),
        pltpu.VMEM((CHUNK, DP), jnp.float32),
        pltpu.VMEM((CHUNK, DP), jnp.float32),
        pltpu.VMEM((CHUNK, DP), jnp.float32),
        pltpu.VMEM((CHUNK, DP), jnp.float32),
        pltpu.VMEM((LANES,), jnp.float32),
        pltpu.SemaphoreType.DMA,
        pltpu.SemaphoreType.DMA,
        pltpu.SemaphoreType.DMA,
    ],
)


def _tc_reduce_body(p_ref, o_ref):
    o_ref[0, 0] = jnp.sum(p_ref[...])


TBLK = 16384


HALF = 524288
LOG2H = 19


def _tc_pairize_body(a_ref, b_ref, o_ref):
    o_ref[:, :D] = jnp.transpose(a_ref[...], (1, 0))
    o_ref[:, D:] = jnp.transpose(b_ref[...], (1, 0))


def _tc_pairize(wt):
    n = wt.shape[1]
    nb = HALF // TBLK
    last_full = n // TBLK - 1
    return pl.pallas_call(
        _tc_pairize_body,
        grid=(nb,),
        in_specs=[pl.BlockSpec((D, TBLK), lambda j: (0, j)),
                  pl.BlockSpec((D, TBLK),
                               lambda j: (0, jnp.minimum(j + nb, last_full)))],
        out_specs=pl.BlockSpec((TBLK, DP), lambda j: (j, 0)),
        out_shape=jax.ShapeDtypeStruct((HALF, DP), jnp.float32),
    )(wt, wt)


def kernel(pos_h, pos_r, pos_t, neg_h, neg_r, neg_t, W_en, W_re):
    del neg_h, neg_r, neg_t
    h2 = pos_h.reshape(IDX_ROWS, IDX_COLS)
    r2 = pos_r.reshape(IDX_ROWS, IDX_COLS)
    t2 = pos_t.reshape(IDX_ROWS, IDX_COLS)
    wen2 = _tc_pairize(W_en.T)
    n = W_en.shape[0]
    row0 = (n // TBLK - 1 - HALF // TBLK + 1) * TBLK
    tail = (n - HALF) - row0
    patch = jnp.concatenate(
        [lax.dynamic_slice_in_dim(W_en, row0, tail, 0),
         lax.dynamic_slice_in_dim(W_en, row0 + HALF, tail, 0)], axis=1)
    wen2 = lax.dynamic_update_slice(wen2, patch, (row0, 0))
    wre2 = W_re.reshape(W_re.shape[0] // 2, DP)
    partials = _sc_gather_reduce(h2, r2, t2, wen2, wre2)
    total = pl.pallas_call(
        _tc_reduce_body,
        out_shape=jax.ShapeDtypeStruct((1, 1), jnp.float32),
        out_specs=pl.BlockSpec(memory_space=pltpu.SMEM),
    )(partials)
    return total[0, 0]


# speedup vs baseline: 3.0203x; 1.2429x over previous
import jax
import jax.numpy as jnp
from jax import lax
from jax.experimental import pallas as pl
from jax.experimental.pallas import tpu as pltpu
from jax.experimental.pallas import tpu_sc as plsc

NC = 2
NS = 16
NW = NC * NS
LANES = 16
BATCH = 16384
D = 64
DP = 128
CHUNK = 128
B_PER_W = BATCH // NW
N_CHUNKS = B_PER_W // CHUNK
IDX_COLS = 128
IDX_ROWS = BATCH // IDX_COLS
ROWS_PER_W = N_CHUNKS
COL_CHUNKS = D // LANES
GROUPS = CHUNK // LANES


def _sc_body(h_hbm, r_hbm, t_hbm, wen_hbm, wre_hbm, out_hbm,
             hidx, ridx, tidx, hpair, rpair, tpair, hoff, roff, toff,
             hbuf0, rbuf0, tbuf0, hbuf1, rbuf1, tbuf1, acc,
             sem_i, sem0, sem1):
    wid = lax.axis_index("s") * NC + lax.axis_index("c")
    acc[...] = jnp.zeros((LANES,), jnp.float32)
    base = wid * ROWS_PER_W
    ci_h = pltpu.async_copy(h_hbm.at[pl.ds(base, ROWS_PER_W)], hidx, sem_i)
    ci_r = pltpu.async_copy(r_hbm.at[pl.ds(base, ROWS_PER_W)], ridx, sem0)
    ci_t = pltpu.async_copy(t_hbm.at[pl.ds(base, ROWS_PER_W)], tidx, sem1)
    ci_h.wait()
    ci_r.wait()
    ci_t.wait()

    @pl.loop(0, ROWS_PER_W)
    def _(j):
        @pl.loop(0, IDX_COLS // LANES)
        def _(q):
            sl = pl.ds(q * LANES, LANES)
            for idx, pair, off in ((hidx, hpair, hoff),
                                   (tidx, tpair, toff)):
                v = idx[j, sl]
                pair[j, sl] = v & (HALF - 1)
                off[j, sl] = (v >> LOG2H) * D
            v = ridx[j, sl]
            rpair[j, sl] = v >> 1
            roff[j, sl] = (v & 1) * D

    bufs = ((hbuf0, rbuf0, tbuf0, sem0), (hbuf1, rbuf1, tbuf1, sem1))

    def fire(j, bset):
        hb, rb, tb, sem = bset
        return (pltpu.async_copy(wen_hbm.at[hpair.at[j]], hb, sem),
                pltpu.async_copy(wre_hbm.at[rpair.at[j]], rb, sem),
                pltpu.async_copy(wen_hbm.at[tpair.at[j]], tb, sem))

    def compute(j, bset):
        hb, rb, tb, _ = bset

        @pl.loop(0, GROUPS)
        def _(g):
            gs = pl.ds(g * LANES, LANES)
            hv = hoff[j, gs]
            rv = roff[j, gs]
            tv = toff[j, gs]
            for k in range(LANES):
                row = g * LANES + k
                ho = hv[k]
                ro = rv[k]
                to = tv[k]
                for c in range(COL_CHUNKS):
                    v = (hb[row, pl.ds(ho + c * LANES, LANES)]
                         + rb[row, pl.ds(ro + c * LANES, LANES)]
                         - tb[row, pl.ds(to + c * LANES, LANES)])
                    acc[...] += v * v

    pending = fire(0, bufs[0])
    for j in range(N_CHUNKS):
        nxt = fire(j + 1, bufs[(j + 1) % 2]) if j + 1 < N_CHUNKS else None
        for cp in pending:
            cp.wait()
        compute(j, bufs[j % 2])
        pending = nxt

    pltpu.sync_copy(acc, out_hbm.at[wid])


_sc_gather_reduce = pl.kernel(
    _sc_body,
    out_type=jax.ShapeDtypeStruct((NW, LANES), jnp.float32),
    mesh=plsc.VectorSubcoreMesh(core_axis_name="c", subcore_axis_name="s"),
    scratch_types=[
        pltpu.VMEM((ROWS_PER_W, IDX_COLS), jnp.int32),
        pltpu.VMEM((ROWS_PER_W, IDX_COLS), jnp.int32),
        pltpu.VMEM((ROWS_PER_W, IDX_COLS), jnp.int32),
        pltpu.VMEM((ROWS_PER_W, IDX_COLS), jnp.int32),
        pltpu.VMEM((ROWS_PER_W, IDX_COLS), jnp.int32),
        pltpu.VMEM((ROWS_PER_W, IDX_COLS), jnp.int32),
        pltpu.VMEM((ROWS_PER_W, IDX_COLS), jnp.int32),
        pltpu.VMEM((ROWS_PER_W, IDX_COLS), jnp.int32),
        pltpu.VMEM((ROWS_PER_W, IDX_COLS), jnp.int32),
        pltpu.VMEM((CHUNK, DP), jnp.float32),
        pltpu.VMEM((CHUNK, DP), jnp.float32),
        pltpu.VMEM((CHUNK, DP), jnp.float32),
        pltpu.VMEM((CHUNK, DP), jnp.float32),
        pltpu.VMEM((CHUNK, DP), jnp.float32),
        pltpu.VMEM((CHUNK, DP), jnp.float32),
        pltpu.VMEM((LANES,), jnp.float32),
        pltpu.SemaphoreType.DMA,
        pltpu.SemaphoreType.DMA,
        pltpu.SemaphoreType.DMA,
    ],
)


def _tc_reduce_body(p_ref, o_ref):
    o_ref[0, 0] = jnp.sum(p_ref[...])


TBLK = 16384


HALF = 524288
LOG2H = 19


def _tc_pairize_body(a_ref, b_ref, o_ref):
    ab = jnp.concatenate([a_ref[...], b_ref[...]], axis=0)
    o_ref[...] = jnp.transpose(ab, (1, 0))


def _tc_pairize(wt):
    n = wt.shape[1]
    nb = HALF // TBLK
    last_full = n // TBLK - 1
    return pl.pallas_call(
        _tc_pairize_body,
        grid=(nb,),
        in_specs=[pl.BlockSpec((D, TBLK), lambda j: (0, j)),
                  pl.BlockSpec((D, TBLK),
                               lambda j: (0, jnp.minimum(j + nb, last_full)))],
        out_specs=pl.BlockSpec((TBLK, DP), lambda j: (j, 0)),
        out_shape=jax.ShapeDtypeStruct((HALF, DP), jnp.float32),
    )(wt, wt)


def kernel(pos_h, pos_r, pos_t, neg_h, neg_r, neg_t, W_en, W_re):
    del neg_h, neg_r, neg_t
    h2 = pos_h.reshape(IDX_ROWS, IDX_COLS)
    r2 = pos_r.reshape(IDX_ROWS, IDX_COLS)
    t2 = pos_t.reshape(IDX_ROWS, IDX_COLS)
    wen2 = _tc_pairize(W_en.T)
    n = W_en.shape[0]
    row0 = (n // TBLK - 1 - HALF // TBLK + 1) * TBLK
    tail = (n - HALF) - row0
    patch = jnp.concatenate(
        [lax.dynamic_slice_in_dim(W_en, row0, tail, 0),
         lax.dynamic_slice_in_dim(W_en, row0 + HALF, tail, 0)], axis=1)
    wen2 = lax.dynamic_update_slice(wen2, patch, (row0, 0))
    wre2 = W_re.reshape(W_re.shape[0] // 2, DP)
    partials = _sc_gather_reduce(h2, r2, t2, wen2, wre2)
    total = pl.pallas_call(
        _tc_reduce_body,
        out_shape=jax.ShapeDtypeStruct((1, 1), jnp.float32),
        out_specs=pl.BlockSpec(memory_space=pltpu.SMEM),
    )(partials)
    return total[0, 0]


# speedup vs baseline: 3.1314x; 1.0368x over previous
import jax
import jax.numpy as jnp
from jax import lax
from jax.experimental import pallas as pl
from jax.experimental.pallas import tpu as pltpu
from jax.experimental.pallas import tpu_sc as plsc

NC = 2
NS = 16
NW = NC * NS
LANES = 16
BATCH = 16384
D = 64
DP = 128
CHUNK = 128
B_PER_W = BATCH // NW
N_CHUNKS = B_PER_W // CHUNK
IDX_COLS = 128
IDX_ROWS = BATCH // IDX_COLS
ROWS_PER_W = N_CHUNKS
COL_CHUNKS = D // LANES
GROUPS = CHUNK // LANES


def _sc_body(h_hbm, r_hbm, t_hbm, wen_hbm, wre_hbm, out_hbm,
             hidx, ridx, tidx, hpair, rpair, tpair, hoff, roff, toff,
             hbuf0, rbuf0, tbuf0, hbuf1, rbuf1, tbuf1, acc,
             sem_i, sem0, sem1):
    wid = lax.axis_index("s") * NC + lax.axis_index("c")
    acc[...] = jnp.zeros((LANES,), jnp.float32)
    base = wid * ROWS_PER_W
    ci_h = pltpu.async_copy(h_hbm.at[pl.ds(base, ROWS_PER_W)], hidx, sem_i)
    ci_r = pltpu.async_copy(r_hbm.at[pl.ds(base, ROWS_PER_W)], ridx, sem0)
    ci_t = pltpu.async_copy(t_hbm.at[pl.ds(base, ROWS_PER_W)], tidx, sem1)
    ci_h.wait()
    ci_r.wait()
    ci_t.wait()

    @pl.loop(0, ROWS_PER_W)
    def _(j):
        @pl.loop(0, IDX_COLS // LANES)
        def _(q):
            sl = pl.ds(q * LANES, LANES)
            for idx, pair, off in ((hidx, hpair, hoff),
                                   (tidx, tpair, toff)):
                v = idx[j, sl]
                pair[j, sl] = v & (HALF - 1)
                off[j, sl] = (v >> LOG2H) * D
            v = ridx[j, sl]
            rpair[j, sl] = v >> 1
            roff[j, sl] = (v & 1) * D

    bufs = ((hbuf0, rbuf0, tbuf0, sem0), (hbuf1, rbuf1, tbuf1, sem1))

    def fire(j, bset):
        hb, rb, tb, sem = bset
        return (pltpu.async_copy(wen_hbm.at[hpair.at[j]], hb, sem),
                pltpu.async_copy(wre_hbm.at[rpair.at[j]], rb, sem),
                pltpu.async_copy(wen_hbm.at[tpair.at[j]], tb, sem))

    def compute(j, bset):
        hb, rb, tb, _ = bset

        @pl.loop(0, GROUPS)
        def _(g):
            gs = pl.ds(g * LANES, LANES)
            hv = hoff[j, gs]
            rv = roff[j, gs]
            tv = toff[j, gs]
            for k in range(LANES):
                row = g * LANES + k
                ho = hv[k]
                ro = rv[k]
                to = tv[k]
                s = None
                for c in range(COL_CHUNKS):
                    v = (hb[row, pl.ds(ho + c * LANES, LANES)]
                         + rb[row, pl.ds(ro + c * LANES, LANES)]
                         - tb[row, pl.ds(to + c * LANES, LANES)])
                    s = v * v if s is None else s + v * v
                acc[...] += s

    pending = fire(0, bufs[0])
    for j in range(N_CHUNKS):
        nxt = fire(j + 1, bufs[(j + 1) % 2]) if j + 1 < N_CHUNKS else None
        for cp in pending:
            cp.wait()
        compute(j, bufs[j % 2])
        pending = nxt

    pltpu.sync_copy(acc, out_hbm.at[wid])


_sc_gather_reduce = pl.kernel(
    _sc_body,
    out_type=jax.ShapeDtypeStruct((NW, LANES), jnp.float32),
    mesh=plsc.VectorSubcoreMesh(core_axis_name="c", subcore_axis_name="s"),
    scratch_types=[
        pltpu.VMEM((ROWS_PER_W, IDX_COLS), jnp.int32),
        pltpu.VMEM((ROWS_PER_W, IDX_COLS), jnp.int32),
        pltpu.VMEM((ROWS_PER_W, IDX_COLS), jnp.int32),
        pltpu.VMEM((ROWS_PER_W, IDX_COLS), jnp.int32),
        pltpu.VMEM((ROWS_PER_W, IDX_COLS), jnp.int32),
        pltpu.VMEM((ROWS_PER_W, IDX_COLS), jnp.int32),
        pltpu.VMEM((ROWS_PER_W, IDX_COLS), jnp.int32),
        pltpu.VMEM((ROWS_PER_W, IDX_COLS), jnp.int32),
        pltpu.VMEM((ROWS_PER_W, IDX_COLS), jnp.int32),
        pltpu.VMEM((CHUNK, DP), jnp.float32),
        pltpu.VMEM((CHUNK, DP), jnp.float32),
        pltpu.VMEM((CHUNK, DP), jnp.float32),
        pltpu.VMEM((CHUNK, DP), jnp.float32),
        pltpu.VMEM((CHUNK, DP), jnp.float32),
        pltpu.VMEM((CHUNK, DP), jnp.float32),
        pltpu.VMEM((LANES,), jnp.float32),
        pltpu.SemaphoreType.DMA,
        pltpu.SemaphoreType.DMA,
        pltpu.SemaphoreType.DMA,
    ],
)


def _tc_reduce_body(p_ref, o_ref):
    o_ref[0, 0] = jnp.sum(p_ref[...])


TBLK = 16384


HALF = 524288
LOG2H = 19


def _tc_pairize_body(a_ref, b_ref, o_ref):
    ab = jnp.concatenate([a_ref[...], b_ref[...]], axis=0)
    o_ref[...] = jnp.transpose(ab, (1, 0))


def _tc_pairize(wt):
    n = wt.shape[1]
    nb = HALF // TBLK
    last_full = n // TBLK - 1
    return pl.pallas_call(
        _tc_pairize_body,
        grid=(nb,),
        in_specs=[pl.BlockSpec((D, TBLK), lambda j: (0, j)),
                  pl.BlockSpec((D, TBLK),
                               lambda j: (0, jnp.minimum(j + nb, last_full)))],
        out_specs=pl.BlockSpec((TBLK, DP), lambda j: (j, 0)),
        out_shape=jax.ShapeDtypeStruct((HALF, DP), jnp.float32),
    )(wt, wt)


def kernel(pos_h, pos_r, pos_t, neg_h, neg_r, neg_t, W_en, W_re):
    del neg_h, neg_r, neg_t
    h2 = pos_h.reshape(IDX_ROWS, IDX_COLS)
    r2 = pos_r.reshape(IDX_ROWS, IDX_COLS)
    t2 = pos_t.reshape(IDX_ROWS, IDX_COLS)
    wen2 = _tc_pairize(W_en.T)
    n = W_en.shape[0]
    row0 = (n // TBLK - 1 - HALF // TBLK + 1) * TBLK
    tail = (n - HALF) - row0
    patch = jnp.concatenate(
        [lax.dynamic_slice_in_dim(W_en, row0, tail, 0),
         lax.dynamic_slice_in_dim(W_en, row0 + HALF, tail, 0)], axis=1)
    wen2 = lax.dynamic_update_slice(wen2, patch, (row0, 0))
    wre2 = W_re.reshape(W_re.shape[0] // 2, DP)
    partials = _sc_gather_reduce(h2, r2, t2, wen2, wre2)
    total = pl.pallas_call(
        _tc_reduce_body,
        out_shape=jax.ShapeDtypeStruct((1, 1), jnp.float32),
        out_specs=pl.BlockSpec(memory_space=pltpu.SMEM),
    )(partials)
    return total[0, 0]
